# Initial kernel scaffold; baseline (speedup 1.0000x reference)
import jax
import jax.numpy as jnp
from jax.experimental import pallas as pl


def kernel(sample_loc, embed_weight, fc_w, fc_b):
    raise NotImplementedError("write your pallas kernel here")


# speedup vs baseline: 629.5829x; 629.5829x over previous
import functools

import jax
import jax.numpy as jnp
from jax import lax
from jax.experimental import pallas as pl
from jax.experimental.pallas import tpu as pltpu
from jax.experimental.pallas import tpu_sc as plsc

BATCH = 4096
NUM_CLASSES = 1000
PAD_CLASSES = 1024
EMBED_DIM = 16

_SC_INFO = plsc.get_sparse_core_info()
_NC = _SC_INFO.num_cores
_NS = _SC_INFO.num_subcores
_NW = _NC * _NS
_BPW = BATCH // _NW
_LANES = 16


def _table_body(ft_ref, e01_ref, b_ref, out_ref):
    acc = jnp.zeros((8, 128), jnp.float32)
    base = b_ref[0, 0]
    for d in range(EMBED_DIM):
        e0d = e01_ref[0, d].astype(jnp.bfloat16).astype(jnp.float32)
        e1d = e01_ref[1, d].astype(jnp.bfloat16).astype(jnp.float32)
        fd = ft_ref[d].astype(jnp.bfloat16).astype(jnp.float32)
        acc = acc + fd * (e1d - e0d)
        base = base + e0d * jnp.sum(fd)
    out_ref[...] = acc + base


def _lookup_kernel(table_hbm, idx_hbm, out_hbm, idx_v, out_v, sem):
    wid = lax.axis_index("s") * _NC + lax.axis_index("c")
    start = wid * _BPW
    pltpu.sync_copy(idx_hbm.at[pl.ds(start, _BPW)], idx_v)
    pltpu.async_copy(table_hbm.at[idx_v], out_v, sem).wait()
    pltpu.sync_copy(out_v, out_hbm.at[pl.ds(start, _BPW)])


@jax.jit
def kernel(sample_loc, embed_weight, fc_w, fc_b):
    fc_flat = fc_w.reshape(-1).astype(jnp.float32)
    fc_pad = jnp.concatenate(
        [fc_flat, jnp.zeros((PAD_CLASSES * EMBED_DIM - fc_flat.shape[0],), jnp.float32)]
    )
    ft = fc_pad.reshape(PAD_CLASSES, EMBED_DIM).T.reshape(EMBED_DIM, 8, 128)
    e01 = embed_weight[0:2].astype(jnp.float32)
    b2d = fc_b.reshape(1, 1).astype(jnp.float32)

    table2d = pl.pallas_call(
        _table_body,
        out_shape=jax.ShapeDtypeStruct((8, 128), jnp.float32),
        in_specs=[
            pl.BlockSpec(memory_space=pltpu.VMEM),
            pl.BlockSpec(memory_space=pltpu.SMEM),
            pl.BlockSpec(memory_space=pltpu.SMEM),
        ],
        out_specs=pl.BlockSpec(memory_space=pltpu.VMEM),
    )(ft, e01, b2d)
    table = table2d.reshape(PAD_CLASSES)

    idx = sample_loc.astype(jnp.int32)

    mesh = plsc.VectorSubcoreMesh(core_axis_name="c", subcore_axis_name="s")
    lookup = functools.partial(
        pl.kernel,
        mesh=mesh,
        out_type=jax.ShapeDtypeStruct((BATCH,), jnp.float32),
        scratch_types=[
            pltpu.VMEM((_BPW,), jnp.int32),
            pltpu.VMEM((_BPW,), jnp.float32),
            pltpu.SemaphoreType.DMA,
        ],
    )(_lookup_kernel)
    out = lookup(table, idx)
    return out.reshape(BATCH, 1)
